# baseline (speedup 1.0000x reference)
import functools

import jax
import jax.numpy as jnp
from jax import lax
from jax.experimental import pallas as pl
from jax.experimental.pallas import tpu as pltpu
from jax.experimental.pallas import tpu_sc as plsc

EPS = 1e-12

B, S, H = 4, 2048, 768
N = B * S
NC, NS = 2, 16
NW = NC * NS
ROWS_PER_TILE = N // NW
CHUNK = 64
NCHUNK = ROWS_PER_TILE // CHUNK

TC_ROWS = S
TC_GRID = N // TC_ROWS


def _sc_gather(word_table, flat_ids, n_rows):
    mesh = plsc.VectorSubcoreMesh(core_axis_name="c", subcore_axis_name="s")
    nchunk = n_rows // (NW * CHUNK)
    ids2d = flat_ids.reshape(NW * nchunk, CHUNK)

    @functools.partial(
        pl.kernel,
        mesh=mesh,
        out_type=jax.ShapeDtypeStruct((n_rows, H), word_table.dtype),
        scratch_types=[
            pltpu.VMEM((nchunk, CHUNK), jnp.int32),
            pltpu.VMEM((CHUNK, H), jnp.float32),
            pltpu.VMEM((CHUNK, H), jnp.float32),
            pltpu.SemaphoreType.DMA,
        ],
    )
    def k(table_hbm, idx_hbm, out_hbm, idx_v, rows_a, rows_b, sem):
        wid = lax.axis_index("s") * NC + lax.axis_index("c")
        pltpu.sync_copy(idx_hbm.at[pl.ds(wid * nchunk, nchunk)], idx_v)
        bufs = (rows_a, rows_b)
        descs = [None] * nchunk
        for c in range(min(2, nchunk)):
            descs[c] = pltpu.async_copy(table_hbm.at[idx_v.at[c]], bufs[c % 2], sem)
        for c in range(nchunk):
            descs[c].wait()
            base = (wid * nchunk + c) * CHUNK
            pltpu.sync_copy(bufs[c % 2], out_hbm.at[pl.ds(base, CHUNK)])
            if c + 2 < nchunk:
                descs[c + 2] = pltpu.async_copy(
                    table_hbm.at[idx_v.at[c + 2]], bufs[c % 2], sem)

    return k(word_table, ids2d)


def _ln_body(g_ref, p_ref, tt_ref, ty_ref, w_ref, b_ref, o_ref):
    tt = tt_ref[...]
    t0 = ty_ref[0:1, :]
    t1 = ty_ref[1:2, :]
    e = g_ref[...] + p_ref[...] + t0 + tt * (t1 - t0)
    u = jnp.mean(e, axis=-1, keepdims=True)
    d = e - u
    s = jnp.mean(d * d, axis=-1, keepdims=True)
    x = d * lax.rsqrt(s + EPS)
    o_ref[...] = w_ref[...] * x + b_ref[...]


def _ln_body_alias(g_ref, p_ref, tt_ref, ty_ref, w_ref, b_ref, prev_ref, o_ref):
    del prev_ref
    _ln_body(g_ref, p_ref, tt_ref, ty_ref, w_ref, b_ref, o_ref)


NSLICE = 4
SEQ_SLICE = B // NSLICE
ROWS_SLICE = SEQ_SLICE * S


def _tc_add_ln(s, gathered_s, pos_table, tt_s, type_table, ln_weight, ln_bias,
               prev):
    specs = [
        pl.BlockSpec((S, H), lambda i: (i, 0)),
        pl.BlockSpec((S, H), lambda i: (0, 0)),
        pl.BlockSpec((S, 1), lambda i: (i, 0)),
        pl.BlockSpec((2, H), lambda i: (0, 0)),
        pl.BlockSpec((1, H), lambda i: (0, 0)),
        pl.BlockSpec((1, H), lambda i: (0, 0)),
    ]
    args = [gathered_s, pos_table, tt_s, type_table, ln_weight, ln_bias]
    if prev is None:
        body = _ln_body
        aliases = {}
    else:
        body = _ln_body_alias
        specs.append(pl.BlockSpec(memory_space=pl.ANY))
        args.append(prev)
        aliases = {6: 0}
    return pl.pallas_call(
        body,
        grid=(SEQ_SLICE,),
        in_specs=specs,
        out_specs=pl.BlockSpec((S, H), lambda i: (s * SEQ_SLICE + i, 0)),
        out_shape=jax.ShapeDtypeStruct((N, H), jnp.float32),
        input_output_aliases=aliases,
        compiler_params=pltpu.CompilerParams(
            dimension_semantics=("arbitrary",)),
    )(*args)


def kernel(input_ids, token_type_ids, word_table, pos_table, type_table,
           ln_weight, ln_bias):
    flat_ids = input_ids.reshape(N).astype(jnp.int32)
    tt_col = token_type_ids.reshape(N, 1).astype(jnp.float32)
    w = ln_weight.reshape(1, H)
    b = ln_bias.reshape(1, H)
    gathered = [
        _sc_gather(word_table,
                   lax.slice_in_dim(flat_ids, s * ROWS_SLICE,
                                    (s + 1) * ROWS_SLICE),
                   ROWS_SLICE)
        for s in range(NSLICE)
    ]
    out = None
    for s in range(NSLICE):
        tt_s = lax.slice_in_dim(tt_col, s * ROWS_SLICE, (s + 1) * ROWS_SLICE)
        out = _tc_add_ln(s, gathered[s], pos_table, tt_s, type_table, w, b,
                         out)
    return out.reshape(B, S, H)


# speedup vs baseline: 1.1494x; 1.1494x over previous
import functools

import jax
import jax.numpy as jnp
from jax import lax
from jax.experimental import pallas as pl
from jax.experimental.pallas import tpu as pltpu
from jax.experimental.pallas import tpu_sc as plsc

EPS = 1e-12

B, S, H = 4, 2048, 768
N = B * S
NC, NS = 2, 16
NW = NC * NS
ROWS_PER_TILE = N // NW
CHUNK = 32

TC_ROWS = S
TC_GRID = N // TC_ROWS


def _sc_gather(word_table, flat_ids, n_rows):
    mesh = plsc.VectorSubcoreMesh(core_axis_name="c", subcore_axis_name="s")
    nchunk = n_rows // (NW * CHUNK)
    ids2d = flat_ids.reshape(NW * nchunk, CHUNK)

    @functools.partial(
        pl.kernel,
        mesh=mesh,
        out_type=jax.ShapeDtypeStruct((n_rows, H), word_table.dtype),
        scratch_types=(
            [pltpu.VMEM((nchunk, CHUNK), jnp.int32)]
            + [pltpu.VMEM((CHUNK, H), jnp.float32) for _ in range(nchunk)]
            + [pltpu.SemaphoreType.DMA, pltpu.SemaphoreType.DMA]
        ),
    )
    def k(table_hbm, idx_hbm, out_hbm, idx_v, *rest):
        bufs = rest[:nchunk]
        gsem, wsem = rest[nchunk], rest[nchunk + 1]
        wid = lax.axis_index("s") * NC + lax.axis_index("c")
        pltpu.sync_copy(idx_hbm.at[pl.ds(wid * nchunk, nchunk)], idx_v)
        gd = [
            pltpu.async_copy(table_hbm.at[idx_v.at[c]], bufs[c], gsem)
            for c in range(nchunk)
        ]
        wd = []
        for c in range(nchunk):
            gd[c].wait()
            base = (wid * nchunk + c) * CHUNK
            wd.append(
                pltpu.async_copy(bufs[c], out_hbm.at[pl.ds(base, CHUNK)], wsem))
        for d in wd:
            d.wait()

    return k(word_table, ids2d)


def _ln_body(g_ref, p_ref, tt_ref, ty_ref, w_ref, b_ref, o_ref):
    tt = tt_ref[...]
    t0 = ty_ref[0:1, :]
    t1 = ty_ref[1:2, :]
    e = g_ref[...] + p_ref[...] + t0 + tt * (t1 - t0)
    u = jnp.mean(e, axis=-1, keepdims=True)
    d = e - u
    s = jnp.mean(d * d, axis=-1, keepdims=True)
    x = d * lax.rsqrt(s + EPS)
    o_ref[...] = w_ref[...] * x + b_ref[...]


def _ln_body_alias(g_ref, p_ref, tt_ref, ty_ref, w_ref, b_ref, prev_ref, o_ref):
    del prev_ref
    _ln_body(g_ref, p_ref, tt_ref, ty_ref, w_ref, b_ref, o_ref)


NSLICE = 2
SEQ_SLICE = B // NSLICE
ROWS_SLICE = SEQ_SLICE * S


def _tc_add_ln(s, gathered_s, pos_table, tt_s, type_table, ln_weight, ln_bias,
               prev):
    specs = [
        pl.BlockSpec((S, H), lambda i: (i, 0)),
        pl.BlockSpec((S, H), lambda i: (0, 0)),
        pl.BlockSpec((S, 1), lambda i: (i, 0)),
        pl.BlockSpec((2, H), lambda i: (0, 0)),
        pl.BlockSpec((1, H), lambda i: (0, 0)),
        pl.BlockSpec((1, H), lambda i: (0, 0)),
    ]
    args = [gathered_s, pos_table, tt_s, type_table, ln_weight, ln_bias]
    if prev is None:
        body = _ln_body
        aliases = {}
    else:
        body = _ln_body_alias
        specs.append(pl.BlockSpec(memory_space=pl.ANY))
        args.append(prev)
        aliases = {6: 0}
    return pl.pallas_call(
        body,
        grid=(SEQ_SLICE,),
        in_specs=specs,
        out_specs=pl.BlockSpec((S, H), lambda i: (s * SEQ_SLICE + i, 0)),
        out_shape=jax.ShapeDtypeStruct((N, H), jnp.float32),
        input_output_aliases=aliases,
        compiler_params=pltpu.CompilerParams(
            dimension_semantics=("arbitrary",)),
    )(*args)


def kernel(input_ids, token_type_ids, word_table, pos_table, type_table,
           ln_weight, ln_bias):
    flat_ids = input_ids.reshape(N).astype(jnp.int32)
    tt_col = token_type_ids.reshape(N, 1).astype(jnp.float32)
    w = ln_weight.reshape(1, H)
    b = ln_bias.reshape(1, H)
    gathered = [
        _sc_gather(word_table,
                   lax.slice_in_dim(flat_ids, s * ROWS_SLICE,
                                    (s + 1) * ROWS_SLICE),
                   ROWS_SLICE)
        for s in range(NSLICE)
    ]
    out = None
    for s in range(NSLICE):
        tt_s = lax.slice_in_dim(tt_col, s * ROWS_SLICE, (s + 1) * ROWS_SLICE)
        out = _tc_add_ln(s, gathered[s], pos_table, tt_s, type_table, w, b,
                         out)
    return out.reshape(B, S, H)
